# baseline (speedup 1.0000x reference)
import functools

import jax
import jax.numpy as jnp
from jax import lax
from jax.experimental import pallas as pl
from jax.experimental.pallas import tpu as pltpu
from jax.experimental.pallas import tpu_sc as plsc

SEQ, BATCH, DIM = 8192, 4, 2048
NC, NS = 2, 16
NW = NC * NS
ROWS_PER_W = SEQ // NW
R = 4
CHUNKS = ROWS_PER_W // R
NBUF_IN = 2


def _sc_body(x_hbm, w_hbm, out_hbm, ybuf, wbuf, obuf, isem0, isem1, osem):
    cid = lax.axis_index("c")
    sid = lax.axis_index("s")
    base = (cid * NS + sid) * ROWS_PER_W
    isems = (isem0, isem1)

    def start_in(chunk, si):
        row0 = base + chunk * R
        pltpu.async_copy(x_hbm.at[pl.ds(row0, R)], ybuf.at[si], isems[si])
        pltpu.async_copy(w_hbm.at[pl.ds(row0, R)], wbuf.at[si], isems[si])

    def wait_in(si):
        pltpu.make_async_copy(x_hbm.at[pl.ds(base, R)], ybuf.at[si], isems[si]).wait()
        pltpu.make_async_copy(w_hbm.at[pl.ds(base, R)], wbuf.at[si], isems[si]).wait()

    def start_out(chunk):
        row0 = base + chunk * R
        pltpu.async_copy(obuf, out_hbm.at[pl.ds(row0, R)], osem)

    def wait_out():
        pltpu.make_async_copy(obuf, out_hbm.at[pl.ds(base, R)], osem).wait()

    def compute(si):
        @plsc.parallel_loop(0, DIM // 16, 1, unroll=16)
        def jbody(j, _si=si):
            off = j * 16
            for r in range(R):
                wv = wbuf[_si, r, pl.ds(off, 16)]
                for b in range(BATCH):
                    obuf[r, b, pl.ds(off, 16)] = (
                        ybuf[_si, r, b, pl.ds(off, 16)] + wv
                    )

    for k in range(NBUF_IN):
        start_in(k, k)

    def outer(g, carry):
        for k in range(NBUF_IN):
            c = g * NBUF_IN + k
            si = k
            wait_in(si)

            @pl.when((g > 0) | (k > 0))
            def _():
                wait_out()

            compute(si)
            start_out(c)

            @pl.when(g < CHUNKS // NBUF_IN - 1)
            def _():
                start_in(c + NBUF_IN, si)
        return carry

    lax.fori_loop(0, CHUNKS // NBUF_IN, outer, 0)
    wait_out()


@functools.partial(
    pl.kernel,
    mesh=plsc.VectorSubcoreMesh(core_axis_name="c", subcore_axis_name="s"),
    out_type=jax.ShapeDtypeStruct((SEQ, BATCH, DIM), jnp.float32),
    scratch_types=[
        pltpu.VMEM((NBUF_IN, R, BATCH, DIM), jnp.float32),
        pltpu.VMEM((NBUF_IN, R, DIM), jnp.float32),
        pltpu.VMEM((R, BATCH, DIM), jnp.float32),
        pltpu.SemaphoreType.DMA,
        pltpu.SemaphoreType.DMA,
        pltpu.SemaphoreType.DMA,
    ],
)
def _sc_add(x_hbm, w_hbm, out_hbm, ybuf, wbuf, obuf, isem0, isem1, osem):
    _sc_body(x_hbm, w_hbm, out_hbm, ybuf, wbuf, obuf, isem0, isem1, osem)


def kernel(x, weight):
    return _sc_add(x, weight[:SEQ])
